# Initial kernel scaffold; baseline (speedup 1.0000x reference)
import jax
import jax.numpy as jnp
from jax.experimental import pallas as pl


def kernel(node_features, edge_index, edge_attr, enc_node_W, enc_node_b, enc_edge_W, enc_edge_b, core_edge_W, core_edge_b, core_node_W, core_node_b, dec_W, dec_b, head1_W, head1_b, head2_W, head2_b, val1_W, val1_b, val2_W, val2_b, val3_W, val3_b):
    raise NotImplementedError("write your pallas kernel here")


# speedup vs baseline: 1.4473x; 1.4473x over previous
import functools
import math

import jax
import jax.numpy as jnp
from jax import lax
from jax.experimental import pallas as pl
from jax.experimental.pallas import tpu as pltpu
from jax.experimental.pallas import tpu_sc as plsc

V = 10000
E = 160000
IN_DIM = 14
ED_DIM = 4
H = 64
N_PASSES = 5

NW = 32
CHUNK = 40
NCHUNK = E // (NW * CHUNK)
VPAD = 10240
ZROWS = 128
EPAD = 128
SROWS = VPAD // NW
SL = 5880
SNCH = SL // CHUNK

_f32 = jnp.float32


def _dot(x, w):
    return jnp.dot(x, w, preferred_element_type=_f32)


def _node_encode(nf, Wn, bn, Ws, Wd):
    R = 2000

    def body(nf_ref, Wn_ref, bn_ref, Ws_ref, Wd_ref, h_ref, a_ref, b_ref):
        h = jnp.maximum(_dot(nf_ref[...], Wn_ref[...]) + bn_ref[...], 0.0)
        h_ref[...] = h
        a_ref[...] = _dot(h, Ws_ref[...])
        b_ref[...] = _dot(h, Wd_ref[...])

    return pl.pallas_call(
        body,
        grid=(V // R,),
        in_specs=[
            pl.BlockSpec((R, IN_DIM), lambda i: (i, 0)),
            pl.BlockSpec((IN_DIM, H), lambda i: (0, 0)),
            pl.BlockSpec((H,), lambda i: (0,)),
            pl.BlockSpec((H, H), lambda i: (0, 0)),
            pl.BlockSpec((H, H), lambda i: (0, 0)),
        ],
        out_specs=[
            pl.BlockSpec((R, H), lambda i: (i, 0)),
            pl.BlockSpec((R, H), lambda i: (i, 0)),
            pl.BlockSpec((R, H), lambda i: (i, 0)),
        ],
        out_shape=[jax.ShapeDtypeStruct((V, H), _f32)] * 3,
    )(nf, Wn, bn, Ws, Wd)


def _edge_encode(ea, We_enc, be_enc, We_core, bias_core):
    R = 8000

    def body(ea_ref, W1_ref, b1_ref, W2_ref, b2_ref, e_ref, ew_ref):
        e = jnp.maximum(_dot(ea_ref[...], W1_ref[...]) + b1_ref[...], 0.0)
        e_ref[...] = e
        ew_ref[...] = _dot(e, W2_ref[...]) + b2_ref[...]

    return pl.pallas_call(
        body,
        grid=(E // R,),
        in_specs=[
            pl.BlockSpec((R, ED_DIM), lambda i: (i, 0)),
            pl.BlockSpec((ED_DIM, H), lambda i: (0, 0)),
            pl.BlockSpec((H,), lambda i: (0,)),
            pl.BlockSpec((H, H), lambda i: (0, 0)),
            pl.BlockSpec((H,), lambda i: (0,)),
        ],
        out_specs=[
            pl.BlockSpec((R, H), lambda i: (i, 0)),
            pl.BlockSpec((R, H), lambda i: (i, 0)),
        ],
        out_shape=[jax.ShapeDtypeStruct((E, H), _f32)] * 2,
    )(ea, We_enc, be_enc, We_core, bias_core)


def _edge_mm(e, We_core, bias_core):
    R = 8000

    def body(e_ref, W_ref, b_ref, ew_ref):
        ew_ref[...] = _dot(e_ref[...], W_ref[...]) + b_ref[...]

    return pl.pallas_call(
        body,
        grid=(E // R,),
        in_specs=[
            pl.BlockSpec((R, H), lambda i: (i, 0)),
            pl.BlockSpec((H, H), lambda i: (0, 0)),
            pl.BlockSpec((H,), lambda i: (0,)),
        ],
        out_specs=pl.BlockSpec((R, H), lambda i: (i, 0)),
        out_shape=jax.ShapeDtypeStruct((E, H), _f32),
    )(e, We_core, bias_core)


def _node_update(h, agg, Wn, bn, Ws, Wd):
    R = 2000

    def body(h_ref, g_ref, Wn_ref, bn_ref, Ws_ref, Wd_ref,
             h2_ref, a_ref, b_ref):
        hcat = jnp.concatenate([h_ref[...], g_ref[...]], axis=-1)
        h2 = jnp.maximum(_dot(hcat, Wn_ref[...]) + bn_ref[...], 0.0)
        h2_ref[...] = h2
        a_ref[...] = _dot(h2, Ws_ref[...])
        b_ref[...] = _dot(h2, Wd_ref[...])

    return pl.pallas_call(
        body,
        grid=(V // R,),
        in_specs=[
            pl.BlockSpec((R, H), lambda i: (i, 0)),
            pl.BlockSpec((R, H), lambda i: (i, 0)),
            pl.BlockSpec((2 * H, H), lambda i: (0, 0)),
            pl.BlockSpec((H,), lambda i: (0,)),
            pl.BlockSpec((H, H), lambda i: (0, 0)),
            pl.BlockSpec((H, H), lambda i: (0, 0)),
        ],
        out_specs=[
            pl.BlockSpec((R, H), lambda i: (i, 0)),
            pl.BlockSpec((R, H), lambda i: (i, 0)),
            pl.BlockSpec((R, H), lambda i: (i, 0)),
        ],
        out_shape=[jax.ShapeDtypeStruct((V, H), _f32)] * 3,
    )(h, agg, Wn, bn, Ws, Wd)


def _decode(h, dec_W, dec_b, h1W, h1b, h2W, h2b, v1W, v1b, v2W, v2b, v3W, v3b):
    R = 2000
    NG = V // R

    def body(h_ref, dW_ref, db_ref, h1W_ref, h1b_ref, h2W_ref, h2b_ref,
             v1W_ref, v1b_ref, v2W_ref, v2b_ref, v3Wt_ref, v3b_ref,
             emb_ref, lg_ref, sc_ref, val_ref, acc_ref):
        i = pl.program_id(0)
        emb = _dot(h_ref[...], dW_ref[...]) + db_ref[...]
        emb_ref[...] = emb
        sh = jnp.maximum(_dot(emb, h1W_ref[...]) + h1b_ref[...], 0.0)
        lg = _dot(sh, h2W_ref[...]) + h2b_ref[...]
        lg_ref[...] = lg
        sc_ref[...] = jax.nn.sigmoid(lg)
        bsum = jnp.sum(emb, axis=0, keepdims=True)

        @pl.when(i == 0)
        def _():
            acc_ref[...] = bsum

        @pl.when(i > 0)
        def _():
            acc_ref[...] = acc_ref[...] + bsum

        @pl.when(i == NG - 1)
        def _():
            bf = jnp.bfloat16
            ge = acc_ref[...] / math.sqrt(float(V))
            v1 = jnp.maximum(
                _dot(ge.astype(bf), v1W_ref[...].astype(bf)) + v1b_ref[...],
                0.0)
            v2 = jnp.maximum(
                _dot(v1.astype(bf), v2W_ref[...].astype(bf)) + v2b_ref[...],
                0.0)
            val_ref[...] = (jnp.sum(v2 * v3Wt_ref[...], axis=1, keepdims=True)
                            + v3b_ref[...])

    full = lambda *s: pl.BlockSpec(s, lambda i: tuple(0 for _ in s))
    return pl.pallas_call(
        body,
        grid=(NG,),
        in_specs=[
            pl.BlockSpec((R, H), lambda i: (i, 0)),
            full(H, H), full(H), full(H, H), full(H), full(H, 1), full(1),
            full(H, 120), full(120), full(120, H), full(H), full(1, H), full(1),
        ],
        out_specs=[
            pl.BlockSpec((R, H), lambda i: (i, 0)),
            pl.BlockSpec((R, 1), lambda i: (i, 0)),
            pl.BlockSpec((R, 1), lambda i: (i, 0)),
            pl.BlockSpec((1, 1), lambda i: (0, 0)),
        ],
        out_shape=[
            jax.ShapeDtypeStruct((V, H), _f32),
            jax.ShapeDtypeStruct((V, 1), _f32),
            jax.ShapeDtypeStruct((V, 1), _f32),
            jax.ShapeDtypeStruct((1, 1), _f32),
        ],
        scratch_shapes=[pltpu.VMEM((1, H), _f32)],
    )(h, dec_W, dec_b, h1W, h1b, h2W, h2b, v1W, v1b, v2W, v2b,
      v3W.reshape(1, H), v3b)


_sc_mesh = plsc.VectorSubcoreMesh(core_axis_name="c", subcore_axis_name="s")


@functools.partial(
    pl.kernel,
    mesh=_sc_mesh,
    compiler_params=pltpu.CompilerParams(use_tc_tiling_on_sc=False),
    out_type=jax.ShapeDtypeStruct((E + EPAD, H), _f32),
    scratch_types=[
        pltpu.VMEM((NCHUNK, CHUNK), jnp.int32),
        pltpu.VMEM((NCHUNK, CHUNK), jnp.int32),
        pltpu.VMEM((CHUNK, H), _f32),
        pltpu.VMEM((CHUNK, H), _f32),
        pltpu.VMEM((CHUNK, H), _f32),
        pltpu.VMEM((CHUNK, H), _f32),
        pltpu.VMEM((ZROWS, H), _f32),
        pltpu.SemaphoreType.DMA,
        pltpu.SemaphoreType.DMA,
    ],
)
def _sc_edge_pass(a_hbm, b_hbm, ew_hbm, src_hbm, dst_hbm,
                  e_new_hbm,
                  src_v, dst_v, ga, gb, ew_v, out_v, zb,
                  sem_a, sem_b):
    c = lax.axis_index("c")
    s = lax.axis_index("s")
    wid = s * 2 + c
    base = wid * (NCHUNK * CHUNK)

    pltpu.sync_copy(src_hbm.at[wid], src_v)
    pltpu.sync_copy(dst_hbm.at[wid], dst_v)

    zero16 = jnp.zeros((16,), _f32)

    @pl.when(wid == 0)
    def _():
        def zrow(i, carry):
            for q in range(H // 16):
                zb[i, pl.ds(q * 16, 16)] = zero16
            return carry

        lax.fori_loop(0, ZROWS, zrow, 0)
        pltpu.sync_copy(zb, e_new_hbm.at[pl.ds(E, EPAD)])

    def chunk(j, carry):
        cp_a = pltpu.async_copy(a_hbm.at[src_v.at[j]], ga, sem_a)
        cp_b = pltpu.async_copy(b_hbm.at[dst_v.at[j]], gb, sem_b)
        pltpu.sync_copy(ew_hbm.at[pl.ds(base + j * CHUNK, CHUNK)], ew_v)
        cp_a.wait()
        cp_b.wait()
        for r in range(CHUNK):
            for q in range(H // 16):
                sl = pl.ds(q * 16, 16)
                out_v[r, sl] = jnp.maximum(ga[r, sl] + gb[r, sl] + ew_v[r, sl],
                                           0.0)
        pltpu.sync_copy(out_v, e_new_hbm.at[pl.ds(base + j * CHUNK, CHUNK)])
        return carry

    lax.fori_loop(0, NCHUNK, chunk, 0)


@functools.partial(
    pl.kernel,
    mesh=_sc_mesh,
    compiler_params=pltpu.CompilerParams(use_tc_tiling_on_sc=False),
    out_type=jax.ShapeDtypeStruct((VPAD, H), _f32),
    scratch_types=[
        pltpu.VMEM((SNCH, CHUNK), jnp.int32),
        pltpu.VMEM((SNCH, CHUNK), jnp.int32),
        pltpu.VMEM((CHUNK, H), _f32),
        pltpu.VMEM_SHARED((VPAD, H), _f32),
        pltpu.SemaphoreType.DMA,
        pltpu.SemaphoreType.DMA,
    ],
)
def _sc_scatter(e_hbm, gidx_hbm, dstr_hbm,
                agg_hbm,
                gidx_v, dstr_v, upd, accum, sem, sem2):
    c = lax.axis_index("c")
    s = lax.axis_index("s")
    wid = s * 2 + c
    row0 = wid * SROWS

    pltpu.sync_copy(gidx_hbm.at[wid], gidx_v)
    pltpu.sync_copy(dstr_hbm.at[wid], dstr_v)

    zero16 = jnp.zeros((16,), _f32)

    def zrow(i, carry):
        for q in range(H // 16):
            upd[i, pl.ds(q * 16, 16)] = zero16
        return carry

    lax.fori_loop(0, CHUNK, zrow, 0)

    def zcopy(k, carry):
        pltpu.sync_copy(upd, accum.at[pl.ds(row0 + k * CHUNK, CHUNK)])
        return carry

    lax.fori_loop(0, SROWS // CHUNK, zcopy, 0)

    def chunk(j, carry):
        pltpu.async_copy(e_hbm.at[gidx_v.at[j]], upd, sem).wait()
        pltpu.async_copy(upd, accum.at[dstr_v.at[j]], sem2, add=True).wait()
        return carry

    lax.fori_loop(0, SNCH, chunk, 0)

    pltpu.sync_copy(accum.at[pl.ds(row0, SROWS)], agg_hbm.at[pl.ds(row0, SROWS)])


def kernel(node_features, edge_index, edge_attr,
           enc_node_W, enc_node_b, enc_edge_W, enc_edge_b,
           core_edge_W, core_edge_b, core_node_W, core_node_b,
           dec_W, dec_b, head1_W, head1_b, head2_W, head2_b,
           val1_W, val1_b, val2_W, val2_b, val3_W, val3_b):
    src_i = edge_index[0].astype(jnp.int32)
    dst_i = edge_index[1].astype(jnp.int32)
    src3 = src_i.reshape(NW, NCHUNK, CHUNK)
    dst3 = dst_i.reshape(NW, NCHUNK, CHUNK)

    key = dst_i // SROWS
    perm = jnp.argsort(key, stable=True).astype(jnp.int32)
    key_s = key[perm]
    dst_s = dst_i[perm]
    starts = jnp.searchsorted(key_s, jnp.arange(NW, dtype=jnp.int32))
    slot = jnp.arange(E, dtype=jnp.int32) - starts[key_s].astype(jnp.int32)
    pad_ids = E + (jnp.arange(SL, dtype=jnp.int32) % EPAD)
    gidx3 = (jnp.broadcast_to(pad_ids, (NW, SL))
             .at[key_s, slot].set(perm).reshape(NW, SNCH, CHUNK))
    pad_dst = (jnp.arange(NW, dtype=jnp.int32) * SROWS)[:, None]
    dstr3 = (jnp.broadcast_to(pad_dst, (NW, SL))
             .at[key_s, slot].set(dst_s).reshape(NW, SNCH, CHUNK))

    Ws = core_edge_W[:H]
    Wd = core_edge_W[H:2 * H]
    We = core_edge_W[2 * H:]

    h, a, b = _node_encode(node_features, enc_node_W, enc_node_b, Ws, Wd)
    e, eW = _edge_encode(edge_attr, enc_edge_W, enc_edge_b, We, core_edge_b)

    for p in range(N_PASSES):
        e = _sc_edge_pass(a, b, eW, src3, dst3)
        agg = _sc_scatter(e, gidx3, dstr3)[:V]
        h, a, b = _node_update(h, agg, core_node_W, core_node_b, Ws, Wd)
        if p < N_PASSES - 1:
            eW = _edge_mm(e, We, core_edge_b)

    emb, lg, sc, val = _decode(h, dec_W, dec_b, head1_W, head1_b,
                               head2_W, head2_b, val1_W, val1_b,
                               val2_W, val2_b, val3_W, val3_b)
    subset_logits = lg.reshape(V)
    subset_scores = sc.reshape(V)
    value = val.reshape(1)
    return (subset_logits, subset_scores, value, emb)
